# baseline (speedup 1.0000x reference)
import jax
import jax.numpy as jnp
from jax import lax
from jax.experimental import pallas as pl
from jax.experimental.pallas import tpu as pltpu
from jax.experimental.pallas import tpu_sc as plsc

_MAX_L = 2
_NWAVE = 8
_N_NODES = 10000
_N_EDGES = 640000
_LANES = 16
_CHUNK = 4000
_NCHUNKS = _N_EDGES // _CHUNK
_NPARTS = 4
_CHUNKS_PER_PART = _NCHUNKS // _NPARTS
_PACK_CHUNKS_PER_TILE = _NCHUNKS // 32


def _sc_pack_body(neigh_hbm, center_hbm, cut_hbm, cartt_hbm, out_hbm,
                  nb0, cb0, ub0, xb0, yb0, zb0,
                  nb1, cb1, ub1, xb1, yb1, zb1, ob, sem0, sem1):
    nc = plsc.get_sparse_core_info().num_cores
    wid = lax.axis_index("s") * nc + lax.axis_index("c")
    bufs = ((nb0, cb0, ub0, xb0, yb0, zb0, sem0),
            (nb1, cb1, ub1, xb1, yb1, zb1, sem1))

    def _pairs(g, buf):
        nb, cb, ub, xb, yb, zb, sem = buf
        off = g * _CHUNK
        return ((neigh_hbm.at[pl.ds(off, _CHUNK)], nb),
                (center_hbm.at[pl.ds(off, _CHUNK)], cb),
                (cut_hbm.at[pl.ds(off, _CHUNK)], ub),
                (cartt_hbm.at[pl.ds(off, _CHUNK)], xb),
                (cartt_hbm.at[pl.ds(_N_EDGES + off, _CHUNK)], yb),
                (cartt_hbm.at[pl.ds(2 * _N_EDGES + off, _CHUNK)], zb))

    def _fire(g, buf):
        for src, dst in _pairs(g, buf):
            pltpu.async_copy(src, dst, buf[6])

    def _drain(g, buf):
        for src, dst in _pairs(g, buf):
            pltpu.make_async_copy(src, dst, buf[6]).wait()

    g_base = wid * _PACK_CHUNKS_PER_TILE
    _fire(g_base, bufs[0])
    for k in range(_PACK_CHUNKS_PER_TILE):
        g = g_base + k
        buf = bufs[k % 2]
        _drain(g, buf)
        if k + 1 < _PACK_CHUNKS_PER_TILE:
            _fire(g + 1, bufs[(k + 1) % 2])
        nb, cb, ub, xb, yb, zb, _ = buf

        @plsc.parallel_loop(0, _CHUNK // _LANES, unroll=2)
        def _vec(i):
            s = pl.ds(i * _LANES, _LANES)
            u = ub[s]
            ob[s] = nb[s] | (cb[s] << 16)
            ob[pl.ds(_CHUNK + i * _LANES, _LANES)] = plsc.bitcast(
                u, jnp.int32)
            ob[pl.ds(2 * _CHUNK + i * _LANES, _LANES)] = plsc.bitcast(
                u * xb[s], jnp.int32)
            ob[pl.ds(3 * _CHUNK + i * _LANES, _LANES)] = plsc.bitcast(
                u * yb[s], jnp.int32)
            ob[pl.ds(4 * _CHUNK + i * _LANES, _LANES)] = plsc.bitcast(
                u * zb[s], jnp.int32)

        pltpu.sync_copy(ob, out_hbm.at[pl.ds(5 * g * _CHUNK, 5 * _CHUNK)])


def _sc_pack(neigh, center, cut, cart_t):
    mesh = plsc.VectorSubcoreMesh(core_axis_name="c", subcore_axis_name="s")
    f = pl.kernel(
        _sc_pack_body,
        out_type=jax.ShapeDtypeStruct((5 * _N_EDGES,), jnp.int32),
        mesh=mesh,
        scratch_types=(
            ([pltpu.VMEM((_CHUNK,), jnp.int32)] * 2
             + [pltpu.VMEM((_CHUNK,), jnp.float32)] * 4) * 2
            + [pltpu.VMEM((5 * _CHUNK,), jnp.int32)]
            + [pltpu.SemaphoreType.DMA] * 2
        ),
        compiler_params=pltpu.CompilerParams(needs_layout_passes=False),
    )
    return f(neigh, center, cut, cart_t.reshape(-1))


def _sc_scatter_body(coeff_hbm, pack_hbm, out_hbm,
                     c0, c1, c2, c3, a0, a1, a2, a3,
                     bp0, bp1, wb0, wb1, sem0, sem1):
    nc = plsc.get_sparse_core_info().num_cores
    wid = lax.axis_index("s") * nc + lax.axis_index("c")

    q = wid & 7
    p = wid >> 3
    is_dis = q < 2
    col_base = jnp.where(is_dis, 4 * q, 9 + 4 * ((q - 2) % 2))
    wrow = jnp.where(is_dis, 1, 2 + (q - 2) // 2)
    ch_base = jnp.where(is_dis, 4 * q, 8 + 8 * ((q - 2) // 2)
                        + 4 * ((q - 2) % 2))
    g0 = p * _CHUNKS_PER_PART

    cols = (c0, c1, c2, c3)
    accs = (a0, a1, a2, a3)
    bufs = ((bp0, wb0, sem0), (bp1, wb1, sem1))

    def _fire(g, bp, wb, sem):
        pltpu.async_copy(pack_hbm.at[pl.ds(5 * g * _CHUNK, _CHUNK)], bp, sem)
        pltpu.async_copy(pack_hbm.at[pl.ds((5 * g + wrow) * _CHUNK, _CHUNK)],
                         wb, sem)

    def _drain(g, bp, wb, sem):
        pltpu.make_async_copy(
            pack_hbm.at[pl.ds(5 * g * _CHUNK, _CHUNK)], bp, sem).wait()
        pltpu.make_async_copy(
            pack_hbm.at[pl.ds((5 * g + wrow) * _CHUNK, _CHUNK)],
            wb, sem).wait()

    _fire(g0, *bufs[0])
    _fire(g0 + 1, *bufs[1])

    for l in range(4):
        pltpu.sync_copy(coeff_hbm.at[pl.ds((col_base + l) * _N_NODES,
                                           _N_NODES)], cols[l])

    @plsc.parallel_loop(0, _N_NODES // _LANES, unroll=8)
    def _zero(i):
        z = jnp.zeros((_LANES,), jnp.float32)
        for l in range(4):
            accs[l][pl.ds(i * _LANES, _LANES)] = z

    def _chunk_pair(gg, _):
        for b in range(2):
            g = g0 + gg * 2 + b
            bp, wb, sem = bufs[b]
            _drain(g, bp, wb, sem)

            @plsc.parallel_loop(0, _CHUNK // _LANES, unroll=4)
            def _vec(i):
                s = pl.ds(i * _LANES, _LANES)
                v = bp[s]
                ni = v & 0xFFFF
                ci = v >> 16
                w = plsc.bitcast(wb[s], jnp.float32)
                for l in range(4):
                    g16 = plsc.load_gather(cols[l], [ni])
                    plsc.addupdate_scatter(accs[l], [ci], w * g16)

            @pl.when(gg * 2 + b + 2 < _CHUNKS_PER_PART)
            def _():
                _fire(g + 2, bp, wb, sem)
        return 0

    lax.fori_loop(0, _CHUNKS_PER_PART // 2, _chunk_pair, 0)

    for l in range(4):
        pltpu.sync_copy(
            accs[l],
            out_hbm.at[pl.ds((p * 32 + ch_base + l) * _N_NODES, _N_NODES)])


def _sc_scatter(coeff_t, pack):
    mesh = plsc.VectorSubcoreMesh(core_axis_name="c", subcore_axis_name="s")
    f = pl.kernel(
        _sc_scatter_body,
        out_type=jax.ShapeDtypeStruct((_NPARTS * 32 * _N_NODES,),
                                      jnp.float32),
        mesh=mesh,
        scratch_types=(
            [pltpu.VMEM((_N_NODES,), jnp.float32)] * 4
            + [pltpu.VMEM((_N_NODES,), jnp.float32)] * 4
            + [pltpu.VMEM((_CHUNK,), jnp.int32)] * 2
            + [pltpu.VMEM((_CHUNK,), jnp.int32)] * 2
            + [pltpu.SemaphoreType.DMA] * 2
        ),
        compiler_params=pltpu.CompilerParams(needs_layout_passes=False),
    )
    return f(coeff_t.reshape(-1), pack).reshape(_NPARTS, 32, _N_NODES)


def _tc_stage2_body(acc_ref, scale_ref, mp_dis_ref, mp_cart_ref,
                    w1t_ref, b1_ref, w2t_ref, b2_ref, out_ref):
    a4 = acc_ref[...]
    acc = a4[0] + a4[1] + a4[2] + a4[3]
    sd = scale_ref[0:1, :]
    sc = scale_ref[1:2, :]

    md = (acc[0:8, :] + mp_dis_ref[...]) * sd
    h = jnp.dot(w1t_ref[...], md,
                preferred_element_type=jnp.float32) + b1_ref[...]
    h = h * (1.0 / (1.0 + jnp.exp(-h)))
    radial = jnp.dot(w2t_ref[...], h,
                     preferred_element_type=jnp.float32) + b2_ref[...]

    mc = (acc[8:32, :] + mp_cart_ref[...]) * sc
    x = mc[0:8, :]
    y = mc[8:16, :]
    z = mc[16:24, :]
    r2 = x * x + y * y + z * z
    s4 = x * y
    s5 = y * z
    s6 = 3.0 * z * z - r2
    s7 = x * z
    s8 = x * x - y * y
    ang2 = s4 * s4 + s5 * s5 + s6 * s6 + s7 * s7 + s8 * s8

    out_ref[0:8, :] = radial
    out_ref[8:16, :] = radial * r2
    out_ref[16:24, :] = radial * ang2


def _tc_stage2(acc, scale, mp_dis_t, mp_cart_f, w1t, b1c, w2t, b2c):
    return pl.pallas_call(
        _tc_stage2_body,
        out_shape=jax.ShapeDtypeStruct((24, _N_NODES), jnp.float32),
    )(acc, scale, mp_dis_t, mp_cart_f, w1t, b1c, w2t, b2c)


@jax.jit
def kernel(cart, cut_distances, iter_coeff, index_center, index_neigh,
           MP_dis, MP_cart, W1, b1, W2, b2):
    n = iter_coeff.shape[0]
    coeff_t = iter_coeff.T

    pack = _sc_pack(index_neigh, index_center, cut_distances,
                    cart.T)
    acc = _sc_scatter(coeff_t, pack)

    scale = jnp.stack([iter_coeff[:, _NWAVE], iter_coeff[:, -1]], axis=0)
    mp_dis_t = MP_dis.T
    mp_cart_f = MP_cart.reshape(24, n)
    dens = _tc_stage2(acc, scale, mp_dis_t, mp_cart_f,
                      W1.T, b1[:, None], W2.T, b2[:, None])
    return dens.reshape(_MAX_L + 1, _NWAVE, n).transpose(2, 0, 1)


# speedup vs baseline: 1.0396x; 1.0396x over previous
import jax
import jax.numpy as jnp
from jax import lax
from jax.experimental import pallas as pl
from jax.experimental.pallas import tpu as pltpu
from jax.experimental.pallas import tpu_sc as plsc

_MAX_L = 2
_NWAVE = 8
_N_NODES = 10000
_N_EDGES = 640000
_LANES = 16
_CHUNK = 4000
_NCHUNKS = _N_EDGES // _CHUNK
_NPARTS = 4
_CHUNKS_PER_PART = _NCHUNKS // _NPARTS
_PACK_CHUNKS_PER_TILE = _NCHUNKS // 32


def _sc_pack_body(neigh_hbm, center_hbm, cut_hbm, cartt_hbm, out_hbm,
                  nb0, cb0, ub0, xb0, yb0, zb0,
                  nb1, cb1, ub1, xb1, yb1, zb1, ob, sem0, sem1):
    nc = plsc.get_sparse_core_info().num_cores
    wid = lax.axis_index("s") * nc + lax.axis_index("c")
    bufs = ((nb0, cb0, ub0, xb0, yb0, zb0, sem0),
            (nb1, cb1, ub1, xb1, yb1, zb1, sem1))

    def _pairs(g, buf):
        nb, cb, ub, xb, yb, zb, sem = buf
        off = g * _CHUNK
        return ((neigh_hbm.at[pl.ds(off, _CHUNK)], nb),
                (center_hbm.at[pl.ds(off, _CHUNK)], cb),
                (cut_hbm.at[pl.ds(off, _CHUNK)], ub),
                (cartt_hbm.at[pl.ds(off, _CHUNK)], xb),
                (cartt_hbm.at[pl.ds(_N_EDGES + off, _CHUNK)], yb),
                (cartt_hbm.at[pl.ds(2 * _N_EDGES + off, _CHUNK)], zb))

    def _fire(g, buf):
        for src, dst in _pairs(g, buf):
            pltpu.async_copy(src, dst, buf[6])

    def _drain(g, buf):
        for src, dst in _pairs(g, buf):
            pltpu.make_async_copy(src, dst, buf[6]).wait()

    g_base = wid * _PACK_CHUNKS_PER_TILE
    _fire(g_base, bufs[0])
    for k in range(_PACK_CHUNKS_PER_TILE):
        g = g_base + k
        buf = bufs[k % 2]
        _drain(g, buf)
        if k + 1 < _PACK_CHUNKS_PER_TILE:
            _fire(g + 1, bufs[(k + 1) % 2])
        nb, cb, ub, xb, yb, zb, _ = buf

        @plsc.parallel_loop(0, _CHUNK // _LANES, unroll=2)
        def _vec(i):
            s = pl.ds(i * _LANES, _LANES)
            u = ub[s]
            ob[s] = nb[s] | (cb[s] << 16)
            ob[pl.ds(_CHUNK + i * _LANES, _LANES)] = plsc.bitcast(
                u, jnp.int32)
            ob[pl.ds(2 * _CHUNK + i * _LANES, _LANES)] = plsc.bitcast(
                u * xb[s], jnp.int32)
            ob[pl.ds(3 * _CHUNK + i * _LANES, _LANES)] = plsc.bitcast(
                u * yb[s], jnp.int32)
            ob[pl.ds(4 * _CHUNK + i * _LANES, _LANES)] = plsc.bitcast(
                u * zb[s], jnp.int32)

        pltpu.sync_copy(ob, out_hbm.at[pl.ds(5 * g * _CHUNK, 5 * _CHUNK)])


def _sc_pack(neigh, center, cut, cart_t):
    mesh = plsc.VectorSubcoreMesh(core_axis_name="c", subcore_axis_name="s")
    f = pl.kernel(
        _sc_pack_body,
        out_type=jax.ShapeDtypeStruct((5 * _N_EDGES,), jnp.int32),
        mesh=mesh,
        scratch_types=(
            ([pltpu.VMEM((_CHUNK,), jnp.int32)] * 2
             + [pltpu.VMEM((_CHUNK,), jnp.float32)] * 4) * 2
            + [pltpu.VMEM((5 * _CHUNK,), jnp.int32)]
            + [pltpu.SemaphoreType.DMA] * 2
        ),
        compiler_params=pltpu.CompilerParams(needs_layout_passes=False),
    )
    return f(neigh, center, cut, cart_t.reshape(-1))


def _sc_scatter_body(coeff_hbm, pack_hbm, out_hbm,
                     c0, c1, c2, c3, a0, a1, a2, a3,
                     bp0, bp1, wb0, wb1, sem0, sem1):
    nc = plsc.get_sparse_core_info().num_cores
    wid = lax.axis_index("s") * nc + lax.axis_index("c")

    q = wid & 7
    p = wid >> 3
    is_dis = q < 2
    col_base = jnp.where(is_dis, 4 * q, 9 + 4 * ((q - 2) % 2))
    wrow = jnp.where(is_dis, 1, 2 + (q - 2) // 2)
    ch_base = jnp.where(is_dis, 4 * q, 8 + 8 * ((q - 2) // 2)
                        + 4 * ((q - 2) % 2))
    g0 = p * _CHUNKS_PER_PART

    cols = (c0, c1, c2, c3)
    accs = (a0, a1, a2, a3)
    bufs = ((bp0, wb0, sem0), (bp1, wb1, sem1))

    def _fire(g, bp, wb, sem):
        pltpu.async_copy(pack_hbm.at[pl.ds(5 * g * _CHUNK, _CHUNK)], bp, sem)
        pltpu.async_copy(pack_hbm.at[pl.ds((5 * g + wrow) * _CHUNK, _CHUNK)],
                         wb, sem)

    def _drain(g, bp, wb, sem):
        pltpu.make_async_copy(
            pack_hbm.at[pl.ds(5 * g * _CHUNK, _CHUNK)], bp, sem).wait()
        pltpu.make_async_copy(
            pack_hbm.at[pl.ds((5 * g + wrow) * _CHUNK, _CHUNK)],
            wb, sem).wait()

    _fire(g0, *bufs[0])
    _fire(g0 + 1, *bufs[1])

    for l in range(4):
        pltpu.sync_copy(coeff_hbm.at[pl.ds((col_base + l) * _N_NODES,
                                           _N_NODES)], cols[l])

    @plsc.parallel_loop(0, _N_NODES // _LANES, unroll=8)
    def _zero(i):
        z = jnp.zeros((_LANES,), jnp.float32)
        for l in range(4):
            accs[l][pl.ds(i * _LANES, _LANES)] = z

    def _chunk_pair(gg, _):
        for b in range(2):
            g = g0 + gg * 2 + b
            bp, wb, sem = bufs[b]
            _drain(g, bp, wb, sem)

            @plsc.parallel_loop(0, _CHUNK // _LANES, unroll=2)
            def _vec(i):
                s = pl.ds(i * _LANES, _LANES)
                v = bp[s]
                ni = v & 0xFFFF
                ci = v >> 16
                w = plsc.bitcast(wb[s], jnp.float32)
                for l in range(4):
                    g16 = plsc.load_gather(cols[l], [ni])
                    plsc.addupdate_scatter(accs[l], [ci], w * g16)

            @pl.when(gg * 2 + b + 2 < _CHUNKS_PER_PART)
            def _():
                _fire(g + 2, bp, wb, sem)
        return 0

    lax.fori_loop(0, _CHUNKS_PER_PART // 2, _chunk_pair, 0)

    for l in range(4):
        pltpu.sync_copy(
            accs[l],
            out_hbm.at[pl.ds((p * 32 + ch_base + l) * _N_NODES, _N_NODES)])


def _sc_scatter(coeff_t, pack):
    mesh = plsc.VectorSubcoreMesh(core_axis_name="c", subcore_axis_name="s")
    f = pl.kernel(
        _sc_scatter_body,
        out_type=jax.ShapeDtypeStruct((_NPARTS * 32 * _N_NODES,),
                                      jnp.float32),
        mesh=mesh,
        scratch_types=(
            [pltpu.VMEM((_N_NODES,), jnp.float32)] * 4
            + [pltpu.VMEM((_N_NODES,), jnp.float32)] * 4
            + [pltpu.VMEM((_CHUNK,), jnp.int32)] * 2
            + [pltpu.VMEM((_CHUNK,), jnp.int32)] * 2
            + [pltpu.SemaphoreType.DMA] * 2
        ),
        compiler_params=pltpu.CompilerParams(needs_layout_passes=False),
    )
    return f(coeff_t.reshape(-1), pack).reshape(_NPARTS, 32, _N_NODES)


def _tc_stage2_body(acc_ref, scale_ref, mp_dis_ref, mp_cart_ref,
                    w1t_ref, b1_ref, w2t_ref, b2_ref, out_ref):
    a4 = acc_ref[...]
    acc = a4[0] + a4[1] + a4[2] + a4[3]
    sd = scale_ref[0:1, :]
    sc = scale_ref[1:2, :]

    md = (acc[0:8, :] + mp_dis_ref[...]) * sd
    h = jnp.dot(w1t_ref[...], md,
                preferred_element_type=jnp.float32) + b1_ref[...]
    h = h * (1.0 / (1.0 + jnp.exp(-h)))
    radial = jnp.dot(w2t_ref[...], h,
                     preferred_element_type=jnp.float32) + b2_ref[...]

    mc = (acc[8:32, :] + mp_cart_ref[...]) * sc
    x = mc[0:8, :]
    y = mc[8:16, :]
    z = mc[16:24, :]
    r2 = x * x + y * y + z * z
    s4 = x * y
    s5 = y * z
    s6 = 3.0 * z * z - r2
    s7 = x * z
    s8 = x * x - y * y
    ang2 = s4 * s4 + s5 * s5 + s6 * s6 + s7 * s7 + s8 * s8

    out_ref[0:8, :] = radial
    out_ref[8:16, :] = radial * r2
    out_ref[16:24, :] = radial * ang2


def _tc_stage2(acc, scale, mp_dis_t, mp_cart_f, w1t, b1c, w2t, b2c):
    return pl.pallas_call(
        _tc_stage2_body,
        out_shape=jax.ShapeDtypeStruct((24, _N_NODES), jnp.float32),
    )(acc, scale, mp_dis_t, mp_cart_f, w1t, b1c, w2t, b2c)


@jax.jit
def kernel(cart, cut_distances, iter_coeff, index_center, index_neigh,
           MP_dis, MP_cart, W1, b1, W2, b2):
    n = iter_coeff.shape[0]
    coeff_t = iter_coeff.T

    pack = _sc_pack(index_neigh, index_center, cut_distances,
                    cart.T)
    acc = _sc_scatter(coeff_t, pack)

    scale = jnp.stack([iter_coeff[:, _NWAVE], iter_coeff[:, -1]], axis=0)
    mp_dis_t = MP_dis.T
    mp_cart_f = MP_cart.reshape(24, n)
    dens = _tc_stage2(acc, scale, mp_dis_t, mp_cart_f,
                      W1.T, b1[:, None], W2.T, b2[:, None])
    return dens.reshape(_MAX_L + 1, _NWAVE, n).transpose(2, 0, 1)
